# Initial kernel scaffold; baseline (speedup 1.0000x reference)
import jax
import jax.numpy as jnp
from jax.experimental import pallas as pl


def kernel(u_table, v_table, pos_u, pos_v, neg_v, length, embedding_dim):
    raise NotImplementedError("write your pallas kernel here")


# speedup vs baseline: 4.8075x; 4.8075x over previous
import functools

import jax
import jax.numpy as jnp
from jax import lax
from jax.experimental import pallas as pl
from jax.experimental.pallas import tpu as pltpu
from jax.experimental.pallas import tpu_sc as plsc


def _make_sc_partials(B, CTX, NEG, D, NW):
    assert D == 64
    BW = B // NW
    CB = 8
    NCH = BW // CB
    UR = CB * CTX
    NR = CB * NEG

    mesh = plsc.VectorSubcoreMesh(core_axis_name="c", subcore_axis_name="s")
    nw = mesh.num_cores * mesh.num_subcores
    assert nw == NW

    @functools.partial(
        pl.kernel,
        mesh=mesh,
        compiler_params=pltpu.CompilerParams(use_tc_tiling_on_sc=False),
        out_type=[
            jax.ShapeDtypeStruct((B * 16,), jnp.float32),
            jax.ShapeDtypeStruct((B * NEG * 16,), jnp.float32),
        ],
        scratch_types=[
            pltpu.VMEM((BW * CTX,), jnp.int32),
            pltpu.VMEM((BW * NEG,), jnp.int32),
            pltpu.VMEM((BW,), jnp.int32),
            pltpu.VMEM((UR, D), jnp.float32),
            pltpu.VMEM((NR, D), jnp.float32),
            pltpu.VMEM((CB, D), jnp.float32),
            pltpu.VMEM((BW * 16,), jnp.float32),
            pltpu.VMEM((NR * 16,), jnp.float32),
            pltpu.SemaphoreType.DMA,
        ],
    )
    def sc_partials(u_hbm, v_hbm, posu_hbm, posv_hbm, negv_hbm,
                    pos_out, neg_out,
                    posu_idx, negv_idx, posv_idx,
                    u_rows, n_rows, pv_rows, pos_part, neg_part, sem):
        wid = lax.axis_index("s") * mesh.num_cores + lax.axis_index("c")
        base = wid * BW
        pltpu.sync_copy(posu_hbm.at[pl.ds(base * CTX, BW * CTX)], posu_idx)
        pltpu.sync_copy(negv_hbm.at[pl.ds(base * NEG, BW * NEG)], negv_idx)
        pltpu.sync_copy(posv_hbm.at[pl.ds(base, BW)], posv_idx)

        def chunk(c, carry):
            cu1 = pltpu.async_copy(
                u_hbm.at[posu_idx.at[pl.ds(c * UR, UR // 2)]],
                u_rows.at[pl.ds(0, UR // 2)], sem)
            cu2 = pltpu.async_copy(
                u_hbm.at[posu_idx.at[pl.ds(c * UR + UR // 2, UR // 2)]],
                u_rows.at[pl.ds(UR // 2, UR // 2)], sem)
            cn1 = pltpu.async_copy(
                v_hbm.at[negv_idx.at[pl.ds(c * NR, NR // 2)]],
                n_rows.at[pl.ds(0, NR // 2)], sem)
            cn2 = pltpu.async_copy(
                v_hbm.at[negv_idx.at[pl.ds(c * NR + NR // 2, NR // 2)]],
                n_rows.at[pl.ds(NR // 2, NR // 2)], sem)
            cv = pltpu.async_copy(
                v_hbm.at[posv_idx.at[pl.ds(c * CB, CB)]], pv_rows, sem)
            cu1.wait(); cu2.wait(); cn1.wait(); cn2.wait(); cv.wait()

            for b in range(CB):
                acc = [u_rows[b * CTX, j * 16:(j + 1) * 16] for j in range(4)]
                for r in range(1, CTX):
                    for j in range(4):
                        acc[j] = acc[j] + u_rows[b * CTX + r, j * 16:(j + 1) * 16]
                t = acc[0] * pv_rows[b, 0:16]
                for j in range(1, 4):
                    t = t + acc[j] * pv_rows[b, j * 16:(j + 1) * 16]
                pos_part[pl.ds((c * CB + b) * 16, 16)] = t
                for n in range(NEG):
                    row = b * NEG + n
                    t2 = acc[0] * n_rows[row, 0:16]
                    for j in range(1, 4):
                        t2 = t2 + acc[j] * n_rows[row, j * 16:(j + 1) * 16]
                    neg_part[row * 16:(row + 1) * 16] = t2
            pltpu.sync_copy(
                neg_part, neg_out.at[pl.ds((base * NEG + c * NR) * 16, NR * 16)])
            return carry

        lax.fori_loop(0, NCH, chunk, 0)
        pltpu.sync_copy(pos_part, pos_out.at[pl.ds(base * 16, BW * 16)])

    return sc_partials


def _make_loss_kernel(n_blocks):
    def loss_kernel(scale_ref, pos_ref, neg_ref, out_ref):
        i = pl.program_id(0)
        inv_len = scale_ref[0]
        rows = lax.broadcasted_iota(jnp.int32, (128, 8), 0)
        cols = lax.broadcasted_iota(jnp.int32, (128, 8), 1)
        fold = jnp.where(rows // 16 == cols, 1.0, 0.0).astype(jnp.float32)

        def logsig(x):
            return jnp.minimum(x, 0.0) - jnp.log1p(jnp.exp(-jnp.abs(x)))

        p = jax.lax.dot(pos_ref[...], fold) * inv_len
        n = jax.lax.dot(neg_ref[...], fold) * inv_len
        part = jnp.sum(logsig(p)) + jnp.sum(logsig(-n))

        @pl.when(i == 0)
        def _():
            out_ref[...] = jnp.zeros((1, 1), jnp.float32)
        out_ref[...] += part[None, None]

    return loss_kernel


def kernel(u_table, v_table, pos_u, pos_v, neg_v, length, embedding_dim):
    B, CTX = pos_u.shape
    NEG = neg_v.shape[1]
    D = u_table.shape[1]
    NW = 32

    posu_flat = pos_u.reshape(-1).astype(jnp.int32)
    negv_flat = neg_v.reshape(-1).astype(jnp.int32)
    posv = pos_v.astype(jnp.int32)

    sc_partials = _make_sc_partials(B, CTX, NEG, D, NW)
    pos_part, neg_part = sc_partials(u_table, v_table, posu_flat, posv, negv_flat)

    pos2d = pos_part.reshape(B * 16 // 128, 128)
    neg2d = neg_part.reshape(B * NEG * 16 // 128, 128)
    GRID = 8
    rp = pos2d.shape[0] // GRID
    rn = neg2d.shape[0] // GRID

    inv_len = (1.0 / jnp.asarray(length, jnp.float32)).reshape(1)

    total = pl.pallas_call(
        _make_loss_kernel(GRID),
        grid=(GRID,),
        in_specs=[
            pl.BlockSpec(memory_space=pltpu.SMEM),
            pl.BlockSpec((rp, 128), lambda i: (i, 0)),
            pl.BlockSpec((rn, 128), lambda i: (i, 0)),
        ],
        out_specs=pl.BlockSpec((1, 1), lambda i: (0, 0)),
        out_shape=jax.ShapeDtypeStruct((1, 1), jnp.float32),
    )(inv_len, pos2d, neg2d)

    return (-total[0, 0]) / jnp.asarray(embedding_dim, jnp.float32)
